# Initial kernel scaffold; baseline (speedup 1.0000x reference)
import jax
import jax.numpy as jnp
from jax.experimental import pallas as pl


def kernel(users, candidates, mask, k, user_table, item_table):
    raise NotImplementedError("write your pallas kernel here")


# speedup vs baseline: 29.4200x; 29.4200x over previous
import functools
import jax
import jax.numpy as jnp
from jax import lax
from jax.experimental import pallas as pl
from jax.experimental.pallas import tpu as pltpu, tpu_sc as plsc

B = 1024
N = 100000
D = 16
NP = 100352
NT = 2048
BM = 256
NW = 32
CPW = NP // NW
GCH = 112
UPW = B // NW
RPW = B // NW
L1N = NP // 16
L1P = 6400
L2P = 512
NSEL = 112
NEG = float("-inf")

_mesh = plsc.VectorSubcoreMesh(core_axis_name="c", subcore_axis_name="s")


def _wid():
    return lax.axis_index("s") * 2 + lax.axis_index("c")


@functools.partial(
    pl.kernel, mesh=_mesh,
    compiler_params=pltpu.CompilerParams(use_tc_tiling_on_sc=False),
    out_type=[jax.ShapeDtypeStruct((B, D), jnp.float32),
              jax.ShapeDtypeStruct((NP, D), jnp.float32)],
    scratch_types=[pltpu.VMEM((GCH,), jnp.int32),
                   pltpu.VMEM((GCH, D), jnp.float32),
                   pltpu.VMEM((UPW,), jnp.int32),
                   pltpu.VMEM((UPW, D), jnp.float32),
                   pltpu.SemaphoreType.DMA],
)
def _sc_gather(users_hbm, cand_hbm, utab_hbm, itab_hbm, u_out, v_out,
               idx_v, rows_v, uidx_v, urows_v, sem):
    w = _wid()
    ub = w * UPW
    pltpu.sync_copy(users_hbm.at[pl.ds(ub, UPW)], uidx_v)
    pltpu.async_copy(utab_hbm.at[uidx_v], urows_v, sem).wait()
    pltpu.sync_copy(urows_v, u_out.at[pl.ds(ub, UPW)])

    def chunk(i, _):
        base = w * CPW + i * GCH
        pltpu.sync_copy(cand_hbm.at[pl.ds(base, GCH)], idx_v)
        pltpu.async_copy(itab_hbm.at[idx_v], rows_v, sem).wait()
        pltpu.sync_copy(rows_v, v_out.at[pl.ds(base, GCH)])
        return _
    lax.fori_loop(0, CPW // GCH, chunk, None)


def _score_body(u_ref, v_ref, m_ref, o_ref):
    nt = pl.program_id(1)
    s = lax.dot_general(u_ref[...], v_ref[...], (((1,), (1,)), ((), ())),
                        preferred_element_type=jnp.float32)
    col = nt * NT + lax.broadcasted_iota(jnp.int32, (BM, NT), 1)
    o_ref[...] = jnp.where((m_ref[...] == 1) & (col < N), s, NEG)


def _tc_scores(u, v, mask):
    return pl.pallas_call(
        _score_body,
        grid=(B // BM, NP // NT),
        in_specs=[
            pl.BlockSpec((BM, D), lambda rb, nt: (rb, 0)),
            pl.BlockSpec((NT, D), lambda rb, nt: (nt, 0)),
            pl.BlockSpec((BM, NT), lambda rb, nt: (rb, nt)),
        ],
        out_specs=pl.BlockSpec((BM, NT), lambda rb, nt: (rb, nt)),
        out_shape=jax.ShapeDtypeStruct((B, NP), jnp.float32),
    )(u, v, mask)


_IOTA = None


def _rmax(v):
    return lax.reduce_max(v, (0,))


def _argl(v, m, iota):
    return lax.reduce_min(jnp.where(v == m, iota, jnp.int32(99)), (0,))


@functools.partial(
    pl.kernel, mesh=_mesh,
    compiler_params=pltpu.CompilerParams(needs_layout_passes=False),
    out_type=[jax.ShapeDtypeStruct((B, NSEL), jnp.int32),
              jax.ShapeDtypeStruct((B, NSEL), jnp.float32)],
    scratch_types=[pltpu.VMEM((NP,), jnp.float32),
                   pltpu.VMEM((L1P,), jnp.float32),
                   pltpu.VMEM((L2P,), jnp.float32),
                   pltpu.VMEM((32,), jnp.float32),
                   pltpu.VMEM((NSEL,), jnp.int32),
                   pltpu.VMEM((NSEL,), jnp.float32),
                   pltpu.VMEM((NSEL,), jnp.int32),
                   pltpu.SemaphoreType.DMA],
)
def _sc_topk(s_hbm, cand_hbm, ids_out, val_out,
             s_v, l1, l2, l3, sel_i, sel_v, ids_v, sem):
    w = _wid()
    iota = lax.iota(jnp.int32, 16)
    fiota = iota.astype(jnp.float32)
    neg16 = jnp.full((16,), NEG, jnp.float32)

    def row_body(r, _):
        row = w * RPW + r
        pltpu.sync_copy(s_hbm.at[row], s_v)

        def l1_body(j, _):
            acc = neg16
            for kk in range(16):
                acc = jnp.maximum(
                    acc, plsc.load_gather(s_v, [j * 256 + iota * 16 + kk]))
            plsc.store_scatter(l1, [j * 16 + iota], acc)
            return _
        lax.fori_loop(0, L1N // 16, l1_body, None)
        for p in range((L1P - L1N) // 16):
            plsc.store_scatter(l1, [L1N + p * 16 + iota], neg16)

        def l2_body(j, _):
            acc = neg16
            for kk in range(16):
                acc = jnp.maximum(
                    acc, plsc.load_gather(l1, [j * 256 + iota * 16 + kk]))
            plsc.store_scatter(l2, [j * 16 + iota], acc)
            return _
        lax.fori_loop(0, L1P // 256, l2_body, None)
        for p in range((L2P - 400) // 16):
            plsc.store_scatter(l2, [400 + p * 16 + iota], neg16)

        for j in range(2):
            acc = neg16
            for kk in range(16):
                acc = jnp.maximum(
                    acc, plsc.load_gather(l2, [j * 256 + iota * 16 + kk]))
            plsc.store_scatter(l3, [j * 16 + iota], acc)

        def blk_body(b, _):
            siv = jnp.zeros((16,), jnp.int32)
            svv = neg16
            for st in range(16):
                va = plsc.load_gather(l3, [iota])
                vb = plsc.load_gather(l3, [16 + iota])
                ma = _rmax(va)
                mb = _rmax(vb)
                use_b = mb > ma
                m3 = jnp.where(use_b, mb, ma)
                vsel = jnp.where(use_b, vb, va)
                l3l = _argl(vsel, m3, iota)
                base3 = jnp.where(use_b, jnp.int32(16), jnp.int32(0))
                g = base3 + l3l
                v2 = plsc.load_gather(l2, [g * 16 + iota])
                l2l = _argl(v2, m3, iota)
                c = g * 16 + l2l
                v1 = plsc.load_gather(l1, [c * 16 + iota])
                l1l = _argl(v1, m3, iota)
                ch = c * 16 + l1l
                vd = plsc.load_gather(s_v, [ch * 16 + iota])
                l0 = _argl(vd, m3, iota)
                jidx = ch * 16 + l0
                siv = jnp.where(iota == st, jidx, siv)
                svv = jnp.where(iota == st, m3, svv)
                vd2 = jnp.where(iota == l0, NEG, vd)
                plsc.store_scatter(s_v, [ch * 16 + iota], vd2)
                m1n = _rmax(vd2)
                v1n = jnp.where(iota == l1l, m1n, v1)
                plsc.store_scatter(l1, [c * 16 + iota], v1n)
                m2n = _rmax(v1n)
                v2n = jnp.where(iota == l2l, m2n, v2)
                plsc.store_scatter(l2, [g * 16 + iota], v2n)
                m3n = _rmax(v2n)
                v3n = jnp.where(iota == l3l, m3n, vsel)
                plsc.store_scatter(l3, [base3 + iota], v3n)
            plsc.store_scatter(sel_i, [b * 16 + iota], siv)
            plsc.store_scatter(sel_v, [b * 16 + iota], svv)
            return _
        lax.fori_loop(0, NSEL // 16, blk_body, None)

        pltpu.async_copy(cand_hbm.at[sel_i], ids_v, sem).wait()
        pltpu.sync_copy(ids_v, ids_out.at[row])
        pltpu.sync_copy(sel_v, val_out.at[row])
        return _
    lax.fori_loop(0, RPW, row_body, None)


def kernel(users, candidates, mask, k, user_table, item_table):
    cand_pad = jnp.concatenate(
        [candidates, jnp.zeros((NP - N,), jnp.int32)])
    u, v = _sc_gather(users, cand_pad, user_table, item_table)
    s = _tc_scores(u, v, mask)
    ids, vals = _sc_topk(s, cand_pad)
    start = k - 100
    return (lax.dynamic_slice_in_dim(ids, start, 100, axis=1),
            lax.dynamic_slice_in_dim(vals, start, 100, axis=1))
